# baseline (speedup 1.0000x reference)
import jax
import jax.numpy as jnp
from jax.experimental import pallas as pl
from jax.experimental.pallas import tpu as pltpu

_B = 32
_V = 1_000_000
_TILE = 8192
_GRID = -(-_V // _TILE)
_NEG = -3.0e38
_IMAX = 2**31 - 1


def _tf_round(x0, x1, r):
    x0 = x0 + x1
    x1 = ((x1 << r) | (x1 >> (32 - r))) ^ x0
    return x0, x1


def _sampler_kernel(z_ref, out_ref, best_ref, bidx_ref):
    step = pl.program_id(0)

    @pl.when(step == 0)
    def _init():
        best_ref[...] = jnp.full((_B, 128), _NEG, jnp.float32)
        bidx_ref[...] = jnp.zeros((_B, 128), jnp.int32)

    v0 = step * _TILE
    col_i = jax.lax.broadcasted_iota(jnp.int32, (_B, _TILE), 1) + v0
    row = jax.lax.broadcasted_iota(jnp.uint32, (_B, _TILE), 0)

    ks1 = jnp.uint32(42)
    ks2 = jnp.uint32(0 ^ 42 ^ 0x1BD11BDA)
    c = row * jnp.uint32(_V) + col_i.astype(jnp.uint32)
    x0 = jnp.zeros_like(c)
    x1 = c + ks1
    _RA = (13, 15, 26, 6)
    _RB = (17, 29, 16, 24)
    for r in _RA:
        x0, x1 = _tf_round(x0, x1, r)
    x0, x1 = x0 + ks1, x1 + (ks2 + 1)
    for r in _RB:
        x0, x1 = _tf_round(x0, x1, r)
    x0, x1 = x0 + ks2, x1 + 2
    for r in _RA:
        x0, x1 = _tf_round(x0, x1, r)
    x0, x1 = x0, x1 + (ks1 + 3)
    for r in _RB:
        x0, x1 = _tf_round(x0, x1, r)
    x0, x1 = x0 + ks1, x1 + (ks2 + 4)
    for r in _RA:
        x0, x1 = _tf_round(x0, x1, r)
    bits = (x0 + ks2) ^ (x1 + 5)

    u = jax.lax.bitcast_convert_type(
        (bits >> 9) | jnp.uint32(0x3F800000), jnp.float32) - 1.0
    tiny = jnp.float32(1.1754943508222875e-38)
    u = jnp.maximum(tiny, u + tiny)
    g = -jnp.log(-jnp.log(u))

    s = z_ref[...] * jnp.float32(0.1) + g
    s = jnp.where(col_i < _V, s, _NEG)

    m = jnp.max(s, axis=1, keepdims=True)
    idx = jnp.min(jnp.where(s == m, col_i, _IMAX), axis=1, keepdims=True)

    best = best_ref[:, 0:1]
    upd = m > best
    new_best = jnp.where(upd, m, best)
    new_idx = jnp.where(upd, idx, bidx_ref[:, 0:1])
    best_ref[...] = jnp.broadcast_to(new_best, (_B, 128))
    bidx_ref[...] = jnp.broadcast_to(new_idx, (_B, 128))

    @pl.when(step == _GRID - 1)
    def _fin():
        out_ref[...] = jnp.broadcast_to(new_idx, (_B, 128))


_CALL = pl.pallas_call(
    _sampler_kernel,
    grid=(_GRID,),
    in_specs=[pl.BlockSpec((_B, _TILE), lambda i: (0, i))],
    out_specs=pl.BlockSpec((_B, 128), lambda i: (0, 0)),
    out_shape=jax.ShapeDtypeStruct((_B, 128), jnp.int32),
    scratch_shapes=[
        pltpu.VMEM((_B, 128), jnp.float32),
        pltpu.VMEM((_B, 128), jnp.int32),
    ],
)


@jax.jit
def _run(logits):
    return _CALL(logits)[:, :1]


def kernel(logits):
    return _run(logits)
